# baseline (speedup 1.0000x reference)
import functools

import jax
import jax.numpy as jnp
from jax import lax
from jax.experimental import pallas as pl
from jax.experimental.pallas import tpu as pltpu
from jax.experimental.pallas import tpu_sc as plsc

_B, _T, _K, _H = 64, 256, 128, 128
_N = _B * _T
_E = 262144
_IN = 2 * _K
_NC, _NS = 2, 16
_CH = 128
_DW = 8
_RPT = _N // _NS


def _sc_degree(dst2d, ones_rows, zeros_rows):
    mesh = plsc.VectorSubcoreMesh(core_axis_name="c", subcore_axis_name="s")

    @functools.partial(
        pl.kernel,
        mesh=mesh,
        out_type=jax.ShapeDtypeStruct((_NC * _N, _DW), jnp.float32),
        scratch_types=[
            pltpu.VMEM((_E // (_NC * _NS * _CH), _CH), jnp.int32),
            pltpu.VMEM((_CH, _DW), jnp.float32),
            pltpu.VMEM_SHARED((_N, _DW), jnp.float32),
            pltpu.SemaphoreType.DMA,
        ],
        compiler_params=pltpu.CompilerParams(use_tc_tiling_on_sc=False),
    )
    def k(dst_hbm, ones_hbm, zeros_hbm, out_hbm, idx_all, ones_v, deg_sh, sem):
        c = lax.axis_index("c")
        s = lax.axis_index("s")
        cpt = _E // (_NC * _NS * _CH)
        chunk0 = (c * _NS + s) * cpt
        pltpu.sync_copy(zeros_hbm, deg_sh.at[pl.ds(s * _RPT, _RPT)])
        pltpu.sync_copy(dst_hbm.at[pl.ds(chunk0, cpt)], idx_all)
        pltpu.sync_copy(ones_hbm, ones_v)
        plsc.subcore_barrier()

        def fire(i, carry):
            pltpu.async_copy(ones_v, deg_sh.at[idx_all.at[i]], sem, add=True)
            return carry

        lax.fori_loop(0, cpt, fire, 0)

        def drain(i, carry):
            pltpu.make_async_copy(ones_v, deg_sh.at[idx_all.at[i]], sem).wait()
            return carry

        lax.fori_loop(0, cpt, drain, 0)
        plsc.subcore_barrier()
        pltpu.sync_copy(deg_sh.at[pl.ds(s * _RPT, _RPT)],
                        out_hbm.at[pl.ds(c * _N + s * _RPT, _RPT)])

    return k(dst2d, ones_rows, zeros_rows)


def _sc_scatter(src2d, dst2d, y_flat, zeros_rows):
    mesh = plsc.VectorSubcoreMesh(core_axis_name="c", subcore_axis_name="s")

    @functools.partial(
        pl.kernel,
        mesh=mesh,
        out_type=jax.ShapeDtypeStruct((_NC * _N, _H // 2), jnp.float32),
        scratch_types=[
            pltpu.VMEM((_E // (_NS * _CH), _CH), jnp.int32),
            pltpu.VMEM((_E // (_NS * _CH), _CH), jnp.int32),
            pltpu.VMEM((4, _CH, _H // 2), jnp.float32),
            pltpu.VMEM_SHARED((_N, _H // 2), jnp.float32),
            pltpu.SemaphoreType.DMA,
            pltpu.SemaphoreType.DMA,
        ],
        compiler_params=pltpu.CompilerParams(use_tc_tiling_on_sc=False),
    )
    def k(src_hbm, dst_hbm, y_hbm, zeros_hbm, out_hbm,
          sidx, didx, rows, z_sh, gsem, ssem):
        c = lax.axis_index("c")
        s = lax.axis_index("s")
        cpt = _E // (_NS * _CH)
        chunk0 = s * cpt
        pltpu.sync_copy(zeros_hbm, z_sh.at[pl.ds(s * _RPT, _RPT)])
        pltpu.sync_copy(src_hbm.at[pl.ds(chunk0, cpt)], sidx)
        pltpu.sync_copy(dst_hbm.at[pl.ds(chunk0, cpt)], didx)
        yoff = c * _N

        def addoff(j, carry):
            for u in range(_CH // 16):
                sl = pl.ds(u * 16, 16)
                sidx[j, sl] = sidx[j, sl] + yoff
            return carry

        lax.fori_loop(0, cpt, addoff, 0)
        plsc.subcore_barrier()

        for kk in range(3):
            pltpu.async_copy(y_hbm.at[sidx.at[kk]], rows.at[kk], gsem)

        def body(i, carry):
            p = lax.rem(i, 4)
            pn = lax.rem(i + 3, 4)
            pltpu.make_async_copy(y_hbm.at[sidx.at[i]], rows.at[p], gsem).wait()
            pltpu.async_copy(rows.at[p], z_sh.at[didx.at[i]], ssem, add=True)

            @pl.when(i > 0)
            def _():
                pltpu.make_async_copy(
                    rows.at[pn], z_sh.at[didx.at[i - 1]], ssem).wait()

            @pl.when(i + 3 < cpt)
            def _():
                pltpu.async_copy(y_hbm.at[sidx.at[i + 3]], rows.at[pn], gsem)

            return carry

        lax.fori_loop(0, cpt, body, 0)
        pltpu.make_async_copy(
            rows.at[(cpt - 1) % 4], z_sh.at[didx.at[cpt - 1]], ssem).wait()
        plsc.subcore_barrier()
        pltpu.sync_copy(z_sh.at[pl.ds(s * _RPT, _RPT)],
                        out_hbm.at[pl.ds(c * _N + s * _RPT, _RPT)])

    return k(src2d, dst2d, y_flat, zeros_rows)


_BLK = 1024


def _xw_body(x_ref, w_ref, o_ref):
    o_ref[...] = jnp.dot(x_ref[...], w_ref[...],
                         preferred_element_type=jnp.float32)


def _tc_xw(x, gcn_W):
    return pl.pallas_call(
        _xw_body,
        grid=(_N // _BLK,),
        in_specs=[
            pl.BlockSpec((_BLK, _IN), lambda i: (i, 0)),
            pl.BlockSpec((_IN, _H), lambda i: (0, 0)),
        ],
        out_specs=pl.BlockSpec((_BLK, _H), lambda i: (i, 0)),
        out_shape=jax.ShapeDtypeStruct((_N, _H), jnp.float32),
    )(x, gcn_W)


def _prep_body(xw_ref, dp_ref, y2_ref, dinv_ref):
    xw = xw_ref[...]
    deg = dp_ref[0, :, 0] + dp_ref[1, :, 0] + 1.0
    dinv = lax.rsqrt(deg)
    y = xw * dinv[:, None]
    y2_ref[0] = y[:, : _H // 2]
    y2_ref[1] = y[:, _H // 2:]
    dinv_ref[...] = jnp.broadcast_to(dinv[:, None], (_BLK, _DW))


def _tc_prep(xw, deg_parts):
    grid = (_N // _BLK,)
    return pl.pallas_call(
        _prep_body,
        grid=grid,
        in_specs=[
            pl.BlockSpec((_BLK, _H), lambda i: (i, 0)),
            pl.BlockSpec((_NC, _BLK, _DW), lambda i: (0, i, 0)),
        ],
        out_specs=[
            pl.BlockSpec((_NC, _BLK, _H // 2), lambda i: (0, i, 0)),
            pl.BlockSpec((_BLK, _DW), lambda i: (i, 0)),
        ],
        out_shape=[
            jax.ShapeDtypeStruct((_NC, _N, _H // 2), jnp.float32),
            jax.ShapeDtypeStruct((_N, _DW), jnp.float32),
        ],
    )(xw, deg_parts)


_UNROLL = 32


def _gru_body(z0_ref, z1_ref, y0_ref, y1_ref, dv_ref, gb_ref, wih_ref,
              bih_ref, whh_ref, bhh_ref, f1_ref, b1_ref, f2_ref, b2_ref,
              f3_ref, b3_ref, f4_ref, b4_ref, out_ref, h_ref, hsum_ref):
    t = pl.program_id(0)

    @pl.when(t == 0)
    def _():
        h_ref[...] = jnp.zeros_like(h_ref)
        hsum_ref[...] = jnp.zeros_like(hsum_ref)

    zz = jnp.concatenate([z0_ref[...], z1_ref[...]], axis=-1)
    yy = jnp.concatenate([y0_ref[...], y1_ref[...]], axis=-1)
    dv = dv_ref[..., 0:1]
    g = dv * (zz + yy) + gb_ref[...]
    gt = jnp.swapaxes(g, 0, 1).reshape(_UNROLL * _B, _H)
    gi_all = jnp.dot(gt, wih_ref[...], preferred_element_type=jnp.float32)
    gi_all = (gi_all + bih_ref[...]).reshape(_UNROLL, _B, 3 * _H)

    h = h_ref[...]
    hs = hsum_ref[...]
    for u in range(_UNROLL):
        gi = gi_all[u]
        gh = jnp.dot(h, whh_ref[...], preferred_element_type=jnp.float32)
        gh = gh + bhh_ref[...]
        i_r, i_z, i_n = gi[:, :_H], gi[:, _H:2 * _H], gi[:, 2 * _H:]
        h_r, h_z, h_n = gh[:, :_H], gh[:, _H:2 * _H], gh[:, 2 * _H:]
        r = jax.nn.sigmoid(i_r + h_r)
        z = jax.nn.sigmoid(i_z + h_z)
        n = jnp.tanh(i_n + r * h_n)
        h = (1.0 - z) * n + z * h
        hs = hs + h
    h_ref[...] = h
    hsum_ref[...] = hs

    @pl.when(t == _T // _UNROLL - 1)
    def _():
        pooled = hsum_ref[...] * (1.0 / _T)
        a = jnp.dot(pooled, f1_ref[...], preferred_element_type=jnp.float32)
        a = jnp.maximum(a + b1_ref[...], 0.0)
        a = jnp.dot(a, f2_ref[...], preferred_element_type=jnp.float32)
        a = jnp.maximum(a + b2_ref[...], 0.0)
        a = jnp.dot(a, f3_ref[...], preferred_element_type=jnp.float32)
        a = jnp.maximum(a + b3_ref[...], 0.0)
        a = jnp.dot(a, f4_ref[...], preferred_element_type=jnp.float32)
        out_ref[...] = a + b4_ref[...]


def _tc_gru(z0r, z1r, y0r, y1r, dinvr, gcn_b2, wihT, bih2,
            whhT, bhh2, f1T, b1, f2T, b2, f3T, b3, f4T, b4):
    def full(shape):
        return pl.BlockSpec(shape, lambda t: tuple(0 for _ in shape))

    half = pl.BlockSpec((_B, _UNROLL, _H // 2), lambda t: (0, t, 0))
    return pl.pallas_call(
        _gru_body,
        grid=(_T // _UNROLL,),
        in_specs=[
            half, half, half, half,
            pl.BlockSpec((_B, _UNROLL, _DW), lambda t: (0, t, 0)),
            full((1, _H)),
            full((_H, 3 * _H)),
            full((1, 3 * _H)),
            full((_H, 3 * _H)),
            full((1, 3 * _H)),
            full((_H, 64)),
            full((1, 64)),
            full((64, 32)),
            full((1, 32)),
            full((32, 16)),
            full((1, 16)),
            full((16, _DW)),
            full((1, _DW)),
        ],
        out_specs=pl.BlockSpec((_B, _DW), lambda t: (0, 0)),
        out_shape=jax.ShapeDtypeStruct((_B, _DW), jnp.float32),
        scratch_shapes=[
            pltpu.VMEM((_B, _H), jnp.float32),
            pltpu.VMEM((_B, _H), jnp.float32),
        ],
        compiler_params=pltpu.CompilerParams(
            dimension_semantics=("arbitrary",)),
    )(z0r, z1r, y0r, y1r, dinvr, gcn_b2, wihT, bih2,
      whhT, bhh2, f1T, b1, f2T, b2, f3T, b3, f4T, b4)


def kernel(keypoints, edge_index, gcn_W, gcn_b, gru_W_ih, gru_W_hh, gru_b_ih,
           gru_b_hh, fc1_W, fc1_b, fc2_W, fc2_b, fc3_W, fc3_b, fc4_W, fc4_b):
    x = keypoints.reshape(_N, _IN)
    ei = edge_index.astype(jnp.int32)
    src2d = ei[0].reshape(_E // _CH, _CH)
    dst2d = ei[1].reshape(_E // _CH, _CH)

    ones_rows = jnp.zeros((_CH, _DW), jnp.float32).at[:, 0].set(1.0)
    zeros_deg = jnp.zeros((_RPT, _DW), jnp.float32)
    zeros_z = jnp.zeros((_RPT, _H // 2), jnp.float32)

    xw = _tc_xw(x, gcn_W)
    deg_parts = _sc_degree(dst2d, ones_rows, zeros_deg).reshape(_NC, _N, _DW)
    y2, dinv8 = _tc_prep(xw, deg_parts)
    z2 = _sc_scatter(src2d, dst2d, y2.reshape(_NC * _N, _H // 2), zeros_z)

    shp = (_B, _T, _H // 2)
    z2 = z2.reshape(_NC, _N, _H // 2)

    f4T = jnp.pad(fc4_W.T, ((0, 0), (0, _DW - 1)))
    b4 = jnp.broadcast_to(fc4_b.reshape(1, 1), (1, _DW))
    out8 = _tc_gru(
        z2[0].reshape(shp), z2[1].reshape(shp),
        y2[0].reshape(shp), y2[1].reshape(shp),
        dinv8.reshape(_B, _T, _DW),
        gcn_b.reshape(1, _H),
        gru_W_ih.T,
        gru_b_ih.reshape(1, 3 * _H),
        gru_W_hh.T, gru_b_hh.reshape(1, 3 * _H),
        fc1_W.T, fc1_b.reshape(1, 64),
        fc2_W.T, fc2_b.reshape(1, 32),
        fc3_W.T, fc3_b.reshape(1, 16),
        f4T, b4,
    )
    return out8[:, :1]


# speedup vs baseline: 1.0499x; 1.0499x over previous
import functools

import jax
import jax.numpy as jnp
from jax import lax
from jax.experimental import pallas as pl
from jax.experimental.pallas import tpu as pltpu
from jax.experimental.pallas import tpu_sc as plsc

_B, _T, _K, _H = 64, 256, 128, 128
_N = _B * _T
_E = 262144
_IN = 2 * _K
_NC, _NS = 2, 16
_CH = 128
_DW = 8
_RPT = _N // _NS


def _sc_degree(dst2d, ones_rows, zeros_rows):
    mesh = plsc.VectorSubcoreMesh(core_axis_name="c", subcore_axis_name="s")

    @functools.partial(
        pl.kernel,
        mesh=mesh,
        out_type=jax.ShapeDtypeStruct((_NC * _N, _DW), jnp.float32),
        scratch_types=[
            pltpu.VMEM((_E // (_NC * _NS * _CH), _CH), jnp.int32),
            pltpu.VMEM((_CH, _DW), jnp.float32),
            pltpu.VMEM_SHARED((_N, _DW), jnp.float32),
            pltpu.SemaphoreType.DMA,
        ],
        compiler_params=pltpu.CompilerParams(use_tc_tiling_on_sc=False),
    )
    def k(dst_hbm, ones_hbm, zeros_hbm, out_hbm, idx_all, ones_v, deg_sh, sem):
        c = lax.axis_index("c")
        s = lax.axis_index("s")
        cpt = _E // (_NC * _NS * _CH)
        chunk0 = (c * _NS + s) * cpt
        pltpu.sync_copy(zeros_hbm, deg_sh.at[pl.ds(s * _RPT, _RPT)])
        pltpu.sync_copy(dst_hbm.at[pl.ds(chunk0, cpt)], idx_all)
        pltpu.sync_copy(ones_hbm, ones_v)
        plsc.subcore_barrier()

        def fire(i, carry):
            pltpu.async_copy(ones_v, deg_sh.at[idx_all.at[i]], sem, add=True)
            return carry

        lax.fori_loop(0, cpt, fire, 0)

        def drain(i, carry):
            pltpu.make_async_copy(ones_v, deg_sh.at[idx_all.at[i]], sem).wait()
            return carry

        lax.fori_loop(0, cpt, drain, 0)
        plsc.subcore_barrier()
        pltpu.sync_copy(deg_sh.at[pl.ds(s * _RPT, _RPT)],
                        out_hbm.at[pl.ds(c * _N + s * _RPT, _RPT)])

    return k(dst2d, ones_rows, zeros_rows)


def _sc_scatter(src2d, dst2d, y_flat, zeros_rows):
    mesh = plsc.VectorSubcoreMesh(core_axis_name="c", subcore_axis_name="s")

    @functools.partial(
        pl.kernel,
        mesh=mesh,
        out_type=jax.ShapeDtypeStruct((_NC * _N, _H // 2), jnp.float32),
        scratch_types=[
            pltpu.VMEM((_E // (_NS * _CH), _CH), jnp.int32),
            pltpu.VMEM((_E // (_NS * _CH), _CH), jnp.int32),
            pltpu.VMEM((4, _CH, _H // 2), jnp.float32),
            pltpu.VMEM_SHARED((_N, _H // 2), jnp.float32),
            pltpu.SemaphoreType.DMA,
            pltpu.SemaphoreType.DMA,
        ],
        compiler_params=pltpu.CompilerParams(use_tc_tiling_on_sc=False),
    )
    def k(src_hbm, dst_hbm, y_hbm, zeros_hbm, out_hbm,
          sidx, didx, rows, z_sh, gsem, ssem):
        c = lax.axis_index("c")
        s = lax.axis_index("s")
        cpt = _E // (_NS * _CH)
        chunk0 = s * cpt
        pltpu.sync_copy(zeros_hbm, z_sh.at[pl.ds(s * _RPT, _RPT)])
        pltpu.sync_copy(src_hbm.at[pl.ds(chunk0, cpt)], sidx)
        pltpu.sync_copy(dst_hbm.at[pl.ds(chunk0, cpt)], didx)
        yoff = c * _N

        def addoff(j, carry):
            for u in range(_CH // 16):
                sl = pl.ds(u * 16, 16)
                sidx[j, sl] = sidx[j, sl] + yoff
            return carry

        lax.fori_loop(0, cpt, addoff, 0)
        plsc.subcore_barrier()

        for kk in range(3):
            pltpu.async_copy(y_hbm.at[sidx.at[kk]], rows.at[kk], gsem)

        def body(i, carry):
            p = lax.rem(i, 4)
            pn = lax.rem(i + 3, 4)
            pltpu.make_async_copy(y_hbm.at[sidx.at[i]], rows.at[p], gsem).wait()
            pltpu.async_copy(rows.at[p], z_sh.at[didx.at[i]], ssem, add=True)

            @pl.when(i > 0)
            def _():
                pltpu.make_async_copy(
                    rows.at[pn], z_sh.at[didx.at[i - 1]], ssem).wait()

            @pl.when(i + 3 < cpt)
            def _():
                pltpu.async_copy(y_hbm.at[sidx.at[i + 3]], rows.at[pn], gsem)

            return carry

        lax.fori_loop(0, cpt, body, 0)
        pltpu.make_async_copy(
            rows.at[(cpt - 1) % 4], z_sh.at[didx.at[cpt - 1]], ssem).wait()
        plsc.subcore_barrier()
        pltpu.sync_copy(z_sh.at[pl.ds(s * _RPT, _RPT)],
                        out_hbm.at[pl.ds(c * _N + s * _RPT, _RPT)])

    return k(src2d, dst2d, y_flat, zeros_rows)


_BLK = 1024


def _prep_body(x_ref, w_ref, dp_ref, y2_ref, dinv_ref):
    xw = jnp.dot(x_ref[...], w_ref[...], preferred_element_type=jnp.float32)
    deg = dp_ref[0, :, 0] + dp_ref[1, :, 0] + 1.0
    dinv = lax.rsqrt(deg)
    y = xw * dinv[:, None]
    y2_ref[0] = y[:, : _H // 2]
    y2_ref[1] = y[:, _H // 2:]
    dinv_ref[...] = jnp.broadcast_to(dinv[:, None], (_BLK, _DW))


def _tc_prep(x, gcn_W, deg_parts):
    grid = (_N // _BLK,)
    return pl.pallas_call(
        _prep_body,
        grid=grid,
        in_specs=[
            pl.BlockSpec((_BLK, _IN), lambda i: (i, 0)),
            pl.BlockSpec((_IN, _H), lambda i: (0, 0)),
            pl.BlockSpec((_NC, _BLK, _DW), lambda i: (0, i, 0)),
        ],
        out_specs=[
            pl.BlockSpec((_NC, _BLK, _H // 2), lambda i: (0, i, 0)),
            pl.BlockSpec((_BLK, _DW), lambda i: (i, 0)),
        ],
        out_shape=[
            jax.ShapeDtypeStruct((_NC, _N, _H // 2), jnp.float32),
            jax.ShapeDtypeStruct((_N, _DW), jnp.float32),
        ],
    )(x, gcn_W, deg_parts)


_UNROLL = 32


def _gru_body(z0_ref, z1_ref, y0_ref, y1_ref, dv_ref, gb_ref, wih_ref,
              bih_ref, whh_ref, bhh_ref, f1_ref, b1_ref, f2_ref, b2_ref,
              f3_ref, b3_ref, f4_ref, b4_ref, out_ref, h_ref, hsum_ref):
    t = pl.program_id(0)

    @pl.when(t == 0)
    def _():
        h_ref[...] = jnp.zeros_like(h_ref)
        hsum_ref[...] = jnp.zeros_like(hsum_ref)

    zz = jnp.concatenate([z0_ref[...], z1_ref[...]], axis=-1)
    yy = jnp.concatenate([y0_ref[...], y1_ref[...]], axis=-1)
    dv = dv_ref[..., 0:1]
    g = dv * (zz + yy) + gb_ref[...]
    gt = jnp.swapaxes(g, 0, 1).reshape(_UNROLL * _B, _H)
    gi_all = jnp.dot(gt, wih_ref[...], preferred_element_type=jnp.float32)
    gi_all = (gi_all + bih_ref[...]).reshape(_UNROLL, _B, 3 * _H)

    h = h_ref[...]
    hs = hsum_ref[...]
    for u in range(_UNROLL):
        gi = gi_all[u]
        gh = jnp.dot(h, whh_ref[...], preferred_element_type=jnp.float32)
        gh = gh + bhh_ref[...]
        i_r, i_z, i_n = gi[:, :_H], gi[:, _H:2 * _H], gi[:, 2 * _H:]
        h_r, h_z, h_n = gh[:, :_H], gh[:, _H:2 * _H], gh[:, 2 * _H:]
        r = jax.nn.sigmoid(i_r + h_r)
        z = jax.nn.sigmoid(i_z + h_z)
        n = jnp.tanh(i_n + r * h_n)
        h = (1.0 - z) * n + z * h
        hs = hs + h
    h_ref[...] = h
    hsum_ref[...] = hs

    @pl.when(t == _T // _UNROLL - 1)
    def _():
        pooled = hsum_ref[...] * (1.0 / _T)
        a = jnp.dot(pooled, f1_ref[...], preferred_element_type=jnp.float32)
        a = jnp.maximum(a + b1_ref[...], 0.0)
        a = jnp.dot(a, f2_ref[...], preferred_element_type=jnp.float32)
        a = jnp.maximum(a + b2_ref[...], 0.0)
        a = jnp.dot(a, f3_ref[...], preferred_element_type=jnp.float32)
        a = jnp.maximum(a + b3_ref[...], 0.0)
        a = jnp.dot(a, f4_ref[...], preferred_element_type=jnp.float32)
        out_ref[...] = a + b4_ref[...]


def _tc_gru(z0r, z1r, y0r, y1r, dinvr, gcn_b2, wihT, bih2,
            whhT, bhh2, f1T, b1, f2T, b2, f3T, b3, f4T, b4):
    def full(shape):
        return pl.BlockSpec(shape, lambda t: tuple(0 for _ in shape))

    half = pl.BlockSpec((_B, _UNROLL, _H // 2), lambda t: (0, t, 0))
    return pl.pallas_call(
        _gru_body,
        grid=(_T // _UNROLL,),
        in_specs=[
            half, half, half, half,
            pl.BlockSpec((_B, _UNROLL, _DW), lambda t: (0, t, 0)),
            full((1, _H)),
            full((_H, 3 * _H)),
            full((1, 3 * _H)),
            full((_H, 3 * _H)),
            full((1, 3 * _H)),
            full((_H, 64)),
            full((1, 64)),
            full((64, 32)),
            full((1, 32)),
            full((32, 16)),
            full((1, 16)),
            full((16, _DW)),
            full((1, _DW)),
        ],
        out_specs=pl.BlockSpec((_B, _DW), lambda t: (0, 0)),
        out_shape=jax.ShapeDtypeStruct((_B, _DW), jnp.float32),
        scratch_shapes=[
            pltpu.VMEM((_B, _H), jnp.float32),
            pltpu.VMEM((_B, _H), jnp.float32),
        ],
        compiler_params=pltpu.CompilerParams(
            dimension_semantics=("arbitrary",)),
    )(z0r, z1r, y0r, y1r, dinvr, gcn_b2, wihT, bih2,
      whhT, bhh2, f1T, b1, f2T, b2, f3T, b3, f4T, b4)


def kernel(keypoints, edge_index, gcn_W, gcn_b, gru_W_ih, gru_W_hh, gru_b_ih,
           gru_b_hh, fc1_W, fc1_b, fc2_W, fc2_b, fc3_W, fc3_b, fc4_W, fc4_b):
    x = keypoints.reshape(_N, _IN)
    ei = edge_index.astype(jnp.int32)
    src2d = ei[0].reshape(_E // _CH, _CH)
    dst2d = ei[1].reshape(_E // _CH, _CH)

    ones_rows = jnp.zeros((_CH, _DW), jnp.float32).at[:, 0].set(1.0)
    zeros_deg = jnp.zeros((_RPT, _DW), jnp.float32)
    zeros_z = jnp.zeros((_RPT, _H // 2), jnp.float32)

    deg_parts = _sc_degree(dst2d, ones_rows, zeros_deg).reshape(_NC, _N, _DW)
    y2, dinv8 = _tc_prep(x, gcn_W, deg_parts)
    z2 = _sc_scatter(src2d, dst2d, y2.reshape(_NC * _N, _H // 2), zeros_z)

    shp = (_B, _T, _H // 2)
    z2 = z2.reshape(_NC, _N, _H // 2)

    f4T = jnp.pad(fc4_W.T, ((0, 0), (0, _DW - 1)))
    b4 = jnp.broadcast_to(fc4_b.reshape(1, 1), (1, _DW))
    out8 = _tc_gru(
        z2[0].reshape(shp), z2[1].reshape(shp),
        y2[0].reshape(shp), y2[1].reshape(shp),
        dinv8.reshape(_B, _T, _DW),
        gcn_b.reshape(1, _H),
        gru_W_ih.T,
        gru_b_ih.reshape(1, 3 * _H),
        gru_W_hh.T, gru_b_hh.reshape(1, 3 * _H),
        fc1_W.T, fc1_b.reshape(1, 64),
        fc2_W.T, fc2_b.reshape(1, 32),
        fc3_W.T, fc3_b.reshape(1, 16),
        f4T, b4,
    )
    return out8[:, :1]
